# baseline (speedup 1.0000x reference)
import functools

import jax
import jax.numpy as jnp
from jax import lax
from jax.experimental import pallas as pl
from jax.experimental.pallas import tpu as pltpu
from jax.experimental.pallas import tpu_sc as plsc

_VOXEL_X = 0.005
_VOXEL_Y = 0.005
_PC_X = 0.0
_PC_Y = 0.0

_B = 4
_NK = 4096
_C = 256
_H = 200
_W = 200
_HW = _H * _W
_L = 16
_NPT = _NK // _L
_NWORK = 32
_WPB = _NWORK // _B
_CPW = _C // _WPB


def _sc_body(bev_hbm, kpx_hbm, kpy_hbm, stride_hbm, out_hbm,
             img_v, kx_v, ky_v, sv_v,
             ia_v, ib_v, ic_v, id_v,
             wa_v, wb_v, wc_v, wd_v, ob_v):
    wid = lax.axis_index("s") * 2 + lax.axis_index("c")
    b = wid // _WPB
    cbase = (wid % _WPB) * _CPW

    pltpu.sync_copy(kpx_hbm.at[b], kx_v)
    pltpu.sync_copy(kpy_hbm.at[b], ky_v)
    pltpu.sync_copy(stride_hbm, sv_v)
    stride = sv_v[...]

    def prep(i, carry):
        sl = pl.ds(i * _L, _L)
        x = ((kx_v[sl] - _PC_X) / _VOXEL_X) / stride
        y = ((ky_v[sl] - _PC_Y) / _VOXEL_Y) / stride
        xt = x.astype(jnp.int32)
        x0 = jnp.where(x < xt.astype(jnp.float32), xt - 1, xt)
        yt = y.astype(jnp.int32)
        y0 = jnp.where(y < yt.astype(jnp.float32), yt - 1, yt)
        x0c = jnp.clip(x0, 0, _W - 1)
        x1c = jnp.clip(x0 + 1, 0, _W - 1)
        y0c = jnp.clip(y0, 0, _H - 1)
        y1c = jnp.clip(y0 + 1, 0, _H - 1)
        x0f = x0c.astype(jnp.float32)
        x1f = x1c.astype(jnp.float32)
        y0f = y0c.astype(jnp.float32)
        y1f = y1c.astype(jnp.float32)
        ia_v[sl] = y0c * _W + x0c
        ib_v[sl] = y1c * _W + x0c
        ic_v[sl] = y0c * _W + x1c
        id_v[sl] = y1c * _W + x1c
        wa_v[sl] = (x1f - x) * (y1f - y)
        wb_v[sl] = (x1f - x) * (y - y0f)
        wc_v[sl] = (x - x0f) * (y1f - y)
        wd_v[sl] = (x - x0f) * (y - y0f)
        return carry

    lax.fori_loop(0, _NPT, prep, 0)

    def chan(cc, carry):
        pltpu.sync_copy(bev_hbm.at[b, cbase + cc], img_v)

        def pt(i, inner):
            sl = pl.ds(i * _L, _L)
            va = plsc.load_gather(img_v, [ia_v[sl]])
            vb = plsc.load_gather(img_v, [ib_v[sl]])
            vc = plsc.load_gather(img_v, [ic_v[sl]])
            vd = plsc.load_gather(img_v, [id_v[sl]])
            ob_v[sl] = (va * wa_v[sl] + vb * wb_v[sl]
                        + vc * wc_v[sl] + vd * wd_v[sl])
            return inner

        lax.fori_loop(0, _NPT, pt, 0)
        pltpu.sync_copy(ob_v, out_hbm.at[b, cbase + cc])
        return carry

    lax.fori_loop(0, _CPW, chan, 0)


_sc_interp = functools.partial(
    pl.kernel,
    mesh=plsc.VectorSubcoreMesh(core_axis_name="c", subcore_axis_name="s"),
    compiler_params=pltpu.CompilerParams(needs_layout_passes=False),
    out_type=jax.ShapeDtypeStruct((_B, _C, _NK), jnp.float32),
    scratch_types=[
        pltpu.VMEM((_HW,), jnp.float32),
        pltpu.VMEM((_NK,), jnp.float32),
        pltpu.VMEM((_NK,), jnp.float32),
        pltpu.VMEM((_L,), jnp.float32),
        pltpu.VMEM((_NK,), jnp.int32),
        pltpu.VMEM((_NK,), jnp.int32),
        pltpu.VMEM((_NK,), jnp.int32),
        pltpu.VMEM((_NK,), jnp.int32),
        pltpu.VMEM((_NK,), jnp.float32),
        pltpu.VMEM((_NK,), jnp.float32),
        pltpu.VMEM((_NK,), jnp.float32),
        pltpu.VMEM((_NK,), jnp.float32),
        pltpu.VMEM((_NK,), jnp.float32),
    ],
)(_sc_body)


def kernel(keypoints, bev_features, bev_stride):
    kpx = keypoints[:, :, 0]
    kpy = keypoints[:, :, 1]
    bev = bev_features.reshape(_B, _C, _HW)
    stride_vec = jnp.full((_L,), bev_stride, jnp.float32)
    out_t = _sc_interp(bev, kpx, kpy, stride_vec)
    return jnp.transpose(out_t, (0, 2, 1))
